# baseline (speedup 1.0000x reference)
import jax
import jax.numpy as jnp
from jax import lax
from jax.experimental import pallas as pl
from jax.experimental.pallas import tpu as pltpu
from jax.experimental.pallas import tpu_sc as plsc

K = 32
L = 16
NC = 2
NS = 16
NW = NC * NS
ROWS = 128
N = 32768
RPW = ROWS // NW
NVEC = N // L
W = 32
NCM = NVEC // W
NSUP = 4
CAND = N + 2 * L

_NINF = float("-inf")

_LEVELS = [(p, 1 << j) for p in range(1, 5) for j in range(p - 1, -1, -1)]


def _ji():
    return lax.iota(jnp.int32, L)


def _perm(v, d):
    return v.at[_ji() ^ d].get(mode="promise_in_bounds")


def _splat(v, l):
    masked = jnp.where(_ji() == l, v, jnp.full((L,), _NINF, jnp.float32))
    return _fold_max(masked)


def _fold_max(v):
    for d in (8, 4, 2, 1):
        v = jnp.maximum(v, _perm(v, d))
    return v


def _fold_sum(v):
    for d in (8, 4, 2, 1):
        v = v + _perm(v, d)
    return v


def _ce_level(v, p, d):
    ji = _ji()
    y = _perm(v, d)
    dlog = d.bit_length() - 1
    wm = (((ji >> dlog) ^ (ji >> p)) & 1) == 0
    return jnp.where(wm, jnp.maximum(v, y), jnp.minimum(v, y))


def _sort16_desc(v):
    for p, d in _LEVELS:
        v = _ce_level(v, p, d)
    return v


def _sort_bitonic16_desc(v):
    for d in (8, 4, 2, 1):
        v = _ce_level(v, 4, d)
    return v


def _rev(v):
    return lax.rev(v, (0,))


def _merge16_desc(a, b):
    b = _rev(b)
    hi = jnp.maximum(a, b)
    lo = jnp.minimum(a, b)
    return _sort_bitonic16_desc(hi), _sort_bitonic16_desc(lo)


def _top32_desc(a0, a1, b0, b1):
    l0 = jnp.maximum(a0, _rev(b1))
    l1 = jnp.maximum(a1, _rev(b0))
    hi = jnp.maximum(l0, l1)
    lo = jnp.minimum(l0, l1)
    return _sort_bitonic16_desc(hi), _sort_bitonic16_desc(lo)


def _merge_chunk(r0, r1, c0, c1):
    s0 = _sort16_desc(c0)
    s1 = _sort16_desc(c1)
    b0, b1 = _merge16_desc(s0, s1)
    return _top32_desc(r0, r1, b0, b1)


def _tree_max(vs):
    while len(vs) > 1:
        vs = [jnp.maximum(vs[i], vs[i + 1]) for i in range(0, len(vs) - 1, 2)] \
            + ([vs[-1]] if len(vs) % 2 else [])
    return vs[0]


def _row_topk(buf, cm, cand):
    ninf = jnp.full((L,), _NINF, jnp.float32)

    def p1_body(i, carry):
        base = i * W * L
        vs = [buf[pl.ds(base + j * L, L)] for j in range(W)]
        cm[pl.ds(i * L, L)] = _tree_max(vs)
        return carry

    lax.fori_loop(0, NCM, p1_body, jnp.int32(0))

    sup = []
    for s in range(NSUP):
        svs = [cm[pl.ds((s * (NCM // NSUP) + j) * L, L)]
               for j in range(NCM // NSUP)]
        sup.append(_tree_max(svs))
    t0, t1 = _merge_chunk(ninf, ninf, sup[0], sup[1])
    t0, t1 = _merge_chunk(t0, t1, sup[2], sup[3])
    thr_s = t1[15]
    thr_vec = jnp.full((L,), thr_s, jnp.float32)

    def cm_body(m, c):
        cmv = cm[pl.ds(m * L, L)]
        hit = _fold_max(cmv)[0] >= thr_s

        def hit_do(c):
            base = m * W * L

            def rc_body(j, cnt):
                b2 = base + j * 4 * L
                for u in range(4):
                    v = buf[pl.ds(b2 + u * L, L)]
                    cnt = cnt + jnp.where(v >= thr_vec, 1.0, 0.0)
                return cnt

            cnt = lax.fori_loop(0, W // 4, rc_body,
                                jnp.full((L,), 0.0, jnp.float32))

            def lane_body(l, c):
                cmv_l = _splat(cmv, l)

                def lane_do(c2):
                    cl = _splat(cnt, l)[0].astype(jnp.int32)

                    def single(c3):
                        cand[pl.ds(c3, L)] = cmv_l
                        return c3 + 1

                    def multi(c3):
                        for h in range(2):
                            def col_body(j, colv, h=h):
                                v = buf[pl.ds(base + (h * L + j) * L, L)]
                                return jnp.where(_ji() == j, _splat(v, l),
                                                 colv)

                            colv = lax.fori_loop(0, L, col_body, ninf)
                            hcnt = _fold_sum(
                                jnp.where(colv >= thr_vec, 1.0, 0.0)
                            )[0].astype(jnp.int32)

                            def emit1(c4, colv=colv):
                                cand[pl.ds(c4, L)] = _fold_max(colv)
                                return c4 + 1

                            def emitn(c4, colv=colv, hcnt=hcnt):
                                cand[pl.ds(c4, L)] = _sort16_desc(colv)
                                return c4 + hcnt

                            def emit(c4, hcnt=hcnt, emit1=emit1,
                                     emitn=emitn):
                                return lax.cond(hcnt == 1, emit1, emitn, c4)

                            c3 = lax.cond(hcnt > 0, emit, lambda c4: c4, c3)
                        return c3

                    return lax.cond(cl == 1, single, multi, c2)

                return lax.cond(cmv_l[0] >= thr_s, lane_do, lambda c2: c2, c)

            return lax.fori_loop(0, L, lane_body, c)

        return lax.cond(hit, hit_do, lambda cc: cc, c)

    cur = lax.fori_loop(0, NCM, cm_body, jnp.int32(0))

    cand[pl.ds(cur, L)] = ninf
    cand[pl.ds(cur + L, L)] = ninf

    nchunks = (cur + 2 * L - 1) // (2 * L)

    def p3_body(c, carry):
        r0, r1 = carry
        c0 = cand[pl.ds(c * 2 * L, L)]
        c1 = cand[pl.ds(c * 2 * L + L, L)]
        return _merge_chunk(r0, r1, c0, c1)

    return lax.fori_loop(0, nchunks, p3_body, (ninf, ninf))


def _sc_body(x_hbm, out_hbm, buf0, buf1, cm, cand, outb, sem0, sem1):
    wid = lax.axis_index("s") * NC + lax.axis_index("c")
    row0 = wid * RPW
    bufs = (buf0, buf1)
    sems = (sem0, sem1)

    pltpu.make_async_copy(x_hbm.at[row0], buf0, sem0).start()
    for r in range(RPW):
        buf, sem = bufs[r % 2], sems[r % 2]
        pltpu.make_async_copy(x_hbm.at[row0 + r], buf, sem).wait()
        if r + 1 < RPW:
            nbuf, nsem = bufs[(r + 1) % 2], sems[(r + 1) % 2]
            pltpu.make_async_copy(x_hbm.at[row0 + r + 1], nbuf, nsem).start()
        t0, t1 = _row_topk(buf, cm, cand)
        outb[r, pl.ds(0, L)] = t0
        outb[r, pl.ds(L, L)] = t1
    pltpu.sync_copy(outb, out_hbm.at[pl.ds(row0, RPW)])


def kernel(x):
    mesh = plsc.VectorSubcoreMesh(
        core_axis_name="c", subcore_axis_name="s", num_cores=NC,
        num_subcores=NS)
    run = pl.kernel(
        _sc_body,
        out_type=jax.ShapeDtypeStruct((ROWS, K), jnp.float32),
        mesh=mesh,
        scratch_types=[
            pltpu.VMEM((N,), jnp.float32),
            pltpu.VMEM((N,), jnp.float32),
            pltpu.VMEM((NCM * L,), jnp.float32),
            pltpu.VMEM((CAND,), jnp.float32),
            pltpu.VMEM((RPW, K), jnp.float32),
            pltpu.SemaphoreType.DMA,
            pltpu.SemaphoreType.DMA,
        ],
    )
    return run(x)


# speedup vs baseline: 2.1008x; 2.1008x over previous
import jax
import jax.numpy as jnp
from jax import lax
from jax.experimental import pallas as pl
from jax.experimental.pallas import tpu as pltpu
from jax.experimental.pallas import tpu_sc as plsc

K = 32
L = 16
NC = 2
NS = 16
NW = NC * NS
ROWS = 128
N = 32768
RPW = ROWS // NW
NVEC = N // L
W = 32
NCM = NVEC // W
NSUP = 4
CAND = N + 2 * L

_NINF = float("-inf")

_LEVELS = [(p, 1 << j) for p in range(1, 5) for j in range(p - 1, -1, -1)]


def _ji():
    return lax.iota(jnp.int32, L)


def _perm(v, d):
    return v.at[_ji() ^ d].get(mode="promise_in_bounds")


def _splat(v, l):
    masked = jnp.where(_ji() == l, v, jnp.full((L,), _NINF, jnp.float32))
    return _fold_max(masked)


def _fold_max(v):
    for d in (8, 4, 2, 1):
        v = jnp.maximum(v, _perm(v, d))
    return v


def _fold_sum(v):
    for d in (8, 4, 2, 1):
        v = v + _perm(v, d)
    return v


def _ce_level(v, p, d):
    ji = _ji()
    y = _perm(v, d)
    dlog = d.bit_length() - 1
    wm = (((ji >> dlog) ^ (ji >> p)) & 1) == 0
    return jnp.where(wm, jnp.maximum(v, y), jnp.minimum(v, y))


def _sort16_desc(v):
    for p, d in _LEVELS:
        v = _ce_level(v, p, d)
    return v


def _sort_bitonic16_desc(v):
    for d in (8, 4, 2, 1):
        v = _ce_level(v, 4, d)
    return v


def _rev(v):
    return lax.rev(v, (0,))


def _merge16_desc(a, b):
    b = _rev(b)
    hi = jnp.maximum(a, b)
    lo = jnp.minimum(a, b)
    return _sort_bitonic16_desc(hi), _sort_bitonic16_desc(lo)


def _top32_desc(a0, a1, b0, b1):
    l0 = jnp.maximum(a0, _rev(b1))
    l1 = jnp.maximum(a1, _rev(b0))
    hi = jnp.maximum(l0, l1)
    lo = jnp.minimum(l0, l1)
    return _sort_bitonic16_desc(hi), _sort_bitonic16_desc(lo)


def _merge_chunk(r0, r1, c0, c1):
    s0 = _sort16_desc(c0)
    s1 = _sort16_desc(c1)
    b0, b1 = _merge16_desc(s0, s1)
    return _top32_desc(r0, r1, b0, b1)


def _tree_max(vs):
    while len(vs) > 1:
        vs = [jnp.maximum(vs[i], vs[i + 1]) for i in range(0, len(vs) - 1, 2)] \
            + ([vs[-1]] if len(vs) % 2 else [])
    return vs[0]


def _max2_chain(vs):
    m, s = vs[0], jnp.full((L,), _NINF, jnp.float32)
    for v in vs[1:]:
        s = jnp.maximum(s, jnp.minimum(m, v))
        m = jnp.maximum(m, v)
    return m, s


def _max2_combine(a, b):
    m1, s1 = a
    m2, s2 = b
    return (jnp.maximum(m1, m2),
            jnp.maximum(jnp.minimum(m1, m2), jnp.maximum(s1, s2)))


def _row_topk(buf, cm, cm2, cand):
    ninf = jnp.full((L,), _NINF, jnp.float32)

    def p1_body(i, carry):
        base = i * W * L
        vs = [buf[pl.ds(base + j * L, L)] for j in range(W)]
        parts = [_max2_chain(vs[q * 8:(q + 1) * 8]) for q in range(W // 8)]
        while len(parts) > 1:
            parts = [_max2_combine(parts[k], parts[k + 1])
                     for k in range(0, len(parts), 2)]
        mx, m2 = parts[0]
        cm[pl.ds(i * L, L)] = mx
        cm2[pl.ds(i * L, L)] = m2
        return carry

    lax.fori_loop(0, NCM, p1_body, jnp.int32(0))

    sup = []
    for s in range(NSUP):
        svs = [cm[pl.ds((s * (NCM // NSUP) + j) * L, L)]
               for j in range(NCM // NSUP)]
        sup.append(_tree_max(svs))
    t0, t1 = _merge_chunk(ninf, ninf, sup[0], sup[1])
    t0, t1 = _merge_chunk(t0, t1, sup[2], sup[3])
    thr_s = t1[15]
    thr_vec = jnp.full((L,), thr_s, jnp.float32)

    pow2 = jnp.left_shift(jnp.full((L,), 1, jnp.int32), _ji())

    def cm_body(m, c):
        cmv = cm[pl.ds(m * L, L)]
        mk = cmv >= thr_vec
        bits0 = _fold_sum(jnp.where(mk, pow2, 0))[0]
        nl = _fold_sum(jnp.where(mk, 1, 0))[0]

        def hit_do(c):
            base = m * W * L
            cm2v = cm2[pl.ds(m * L, L)]

            def lane_do(l, c2):
                def single(c3):
                    cand[pl.ds(c3, L)] = _splat(cmv, l)
                    return c3 + 1

                def multi(c3):
                    for h in range(2):
                        def col_body(j, colv, h=h):
                            v = buf[pl.ds(base + (h * L + j) * L, L)]
                            return jnp.where(_ji() == j, _splat(v, l), colv)

                        colv = lax.fori_loop(0, L, col_body, ninf)
                        hcnt = _fold_sum(
                            jnp.where(colv >= thr_vec, 1.0, 0.0)
                        )[0].astype(jnp.int32)

                        def emit1(c4, colv=colv):
                            cand[pl.ds(c4, L)] = _fold_max(colv)
                            return c4 + 1

                        def emitn(c4, colv=colv, hcnt=hcnt):
                            cand[pl.ds(c4, L)] = _sort16_desc(colv)
                            return c4 + hcnt

                        def emit(c4, hcnt=hcnt, emit1=emit1, emitn=emitn):
                            return lax.cond(hcnt == 1, emit1, emitn, c4)

                        c3 = lax.cond(hcnt > 0, emit, lambda c4: c4, c3)
                    return c3

                is_multi = _splat(cm2v, l)[0] >= thr_s
                return lax.cond(is_multi, multi, single, c2)

            def lane_iter(i, st):
                bits, c2 = st
                b = bits & (-bits)
                l = (lax.bitcast_convert_type(
                    b.astype(jnp.float32), jnp.int32) >> 23) - 127
                return (bits ^ b, lane_do(l, c2))

            return lax.fori_loop(0, nl, lane_iter, (bits0, c))[1]

        return lax.cond(bits0 != 0, hit_do, lambda cc: cc, c)

    cur = lax.fori_loop(0, NCM, cm_body, jnp.int32(0))

    cand[pl.ds(cur, L)] = ninf
    cand[pl.ds(cur + L, L)] = ninf

    nchunks = (cur + 2 * L - 1) // (2 * L)

    def p3_body(c, carry):
        r0, r1 = carry
        c0 = cand[pl.ds(c * 2 * L, L)]
        c1 = cand[pl.ds(c * 2 * L + L, L)]
        return _merge_chunk(r0, r1, c0, c1)

    return lax.fori_loop(0, nchunks, p3_body, (ninf, ninf))


def _sc_body(x_hbm, out_hbm, buf0, buf1, cm, cm2, cand, outb, sem0, sem1):
    wid = lax.axis_index("s") * NC + lax.axis_index("c")
    row0 = wid * RPW
    bufs = (buf0, buf1)
    sems = (sem0, sem1)

    pltpu.make_async_copy(x_hbm.at[row0], buf0, sem0).start()
    for r in range(RPW):
        buf, sem = bufs[r % 2], sems[r % 2]
        pltpu.make_async_copy(x_hbm.at[row0 + r], buf, sem).wait()
        if r + 1 < RPW:
            nbuf, nsem = bufs[(r + 1) % 2], sems[(r + 1) % 2]
            pltpu.make_async_copy(x_hbm.at[row0 + r + 1], nbuf, nsem).start()
        t0, t1 = _row_topk(buf, cm, cm2, cand)
        outb[r, pl.ds(0, L)] = t0
        outb[r, pl.ds(L, L)] = t1
    pltpu.sync_copy(outb, out_hbm.at[pl.ds(row0, RPW)])


def kernel(x):
    mesh = plsc.VectorSubcoreMesh(
        core_axis_name="c", subcore_axis_name="s", num_cores=NC,
        num_subcores=NS)
    run = pl.kernel(
        _sc_body,
        out_type=jax.ShapeDtypeStruct((ROWS, K), jnp.float32),
        mesh=mesh,
        scratch_types=[
            pltpu.VMEM((N,), jnp.float32),
            pltpu.VMEM((N,), jnp.float32),
            pltpu.VMEM((NCM * L,), jnp.float32),
            pltpu.VMEM((NCM * L,), jnp.float32),
            pltpu.VMEM((CAND,), jnp.float32),
            pltpu.VMEM((RPW, K), jnp.float32),
            pltpu.SemaphoreType.DMA,
            pltpu.SemaphoreType.DMA,
        ],
    )
    return run(x)


# speedup vs baseline: 2.1768x; 1.0362x over previous
import jax
import jax.numpy as jnp
from jax import lax
from jax.experimental import pallas as pl
from jax.experimental.pallas import tpu as pltpu
from jax.experimental.pallas import tpu_sc as plsc

K = 32
L = 16
NC = 2
NS = 16
NW = NC * NS
ROWS = 128
N = 32768
RPW = ROWS // NW
NVEC = N // L
W = 32
NCM = NVEC // W
NSUP = 4
CAND = N + 2 * L

_NINF = float("-inf")

_LEVELS = [(p, 1 << j) for p in range(1, 5) for j in range(p - 1, -1, -1)]


def _ji():
    return lax.iota(jnp.int32, L)


def _perm(v, d):
    return v.at[_ji() ^ d].get(mode="promise_in_bounds")


def _splat(v, l):
    masked = jnp.where(_ji() == l, v, jnp.full((L,), _NINF, jnp.float32))
    return _fold_max(masked)


def _fold_max(v):
    for d in (8, 4, 2, 1):
        v = jnp.maximum(v, _perm(v, d))
    return v


def _fold_sum(v):
    for d in (8, 4, 2, 1):
        v = v + _perm(v, d)
    return v


def _ce_level(v, p, d):
    ji = _ji()
    y = _perm(v, d)
    dlog = d.bit_length() - 1
    wm = (((ji >> dlog) ^ (ji >> p)) & 1) == 0
    return jnp.where(wm, jnp.maximum(v, y), jnp.minimum(v, y))


def _sort16_desc(v):
    for p, d in _LEVELS:
        v = _ce_level(v, p, d)
    return v


def _sort_bitonic16_desc(v):
    for d in (8, 4, 2, 1):
        v = _ce_level(v, 4, d)
    return v


def _rev(v):
    return lax.rev(v, (0,))


def _merge16_desc(a, b):
    b = _rev(b)
    hi = jnp.maximum(a, b)
    lo = jnp.minimum(a, b)
    return _sort_bitonic16_desc(hi), _sort_bitonic16_desc(lo)


def _top32_desc(a0, a1, b0, b1):
    l0 = jnp.maximum(a0, _rev(b1))
    l1 = jnp.maximum(a1, _rev(b0))
    hi = jnp.maximum(l0, l1)
    lo = jnp.minimum(l0, l1)
    return _sort_bitonic16_desc(hi), _sort_bitonic16_desc(lo)


def _merge_chunk(r0, r1, c0, c1):
    s0 = _sort16_desc(c0)
    s1 = _sort16_desc(c1)
    b0, b1 = _merge16_desc(s0, s1)
    return _top32_desc(r0, r1, b0, b1)


def _tree_max(vs):
    while len(vs) > 1:
        vs = [jnp.maximum(vs[i], vs[i + 1]) for i in range(0, len(vs) - 1, 2)] \
            + ([vs[-1]] if len(vs) % 2 else [])
    return vs[0]


def _max2_chain(vs):
    m, s = vs[0], jnp.full((L,), _NINF, jnp.float32)
    for v in vs[1:]:
        s = jnp.maximum(s, jnp.minimum(m, v))
        m = jnp.maximum(m, v)
    return m, s


def _max2_combine(a, b):
    m1, s1 = a
    m2, s2 = b
    return (jnp.maximum(m1, m2),
            jnp.maximum(jnp.minimum(m1, m2), jnp.maximum(s1, s2)))


def _row_topk(buf, cm, cm2, cand):
    ninf = jnp.full((L,), _NINF, jnp.float32)

    @plsc.parallel_loop(0, NCM, 1, unroll=2)
    def _p1(i):
        base = i * W * L
        vs = [buf[pl.ds(base + j * L, L)] for j in range(W)]
        parts = [_max2_chain(vs[q * 8:(q + 1) * 8]) for q in range(W // 8)]
        while len(parts) > 1:
            parts = [_max2_combine(parts[k], parts[k + 1])
                     for k in range(0, len(parts), 2)]
        mx, m2 = parts[0]
        cm[pl.ds(i * L, L)] = mx
        cm2[pl.ds(i * L, L)] = m2

    sup = []
    for s in range(NSUP):
        svs = [cm[pl.ds((s * (NCM // NSUP) + j) * L, L)]
               for j in range(NCM // NSUP)]
        sup.append(_tree_max(svs))
    t0, t1 = _merge_chunk(ninf, ninf, sup[0], sup[1])
    t0, t1 = _merge_chunk(t0, t1, sup[2], sup[3])
    thr_s = t1[15]
    thr_vec = jnp.full((L,), thr_s, jnp.float32)

    pow2 = jnp.left_shift(jnp.full((L,), 1, jnp.int32), _ji())

    def cm_body(m, c):
        cmv = cm[pl.ds(m * L, L)]
        mk = cmv >= thr_vec
        bits0 = _fold_sum(jnp.where(mk, pow2, 0))[0]
        nl = _fold_sum(jnp.where(mk, 1, 0))[0]

        def hit_do(c):
            base = m * W * L
            cm2v = cm2[pl.ds(m * L, L)]

            def lane_do(l, c2):
                def single(c3):
                    cand[pl.ds(c3, L)] = _splat(cmv, l)
                    return c3 + 1

                def multi(c3):
                    for h in range(2):
                        def col_body(j, colv, h=h):
                            v = buf[pl.ds(base + (h * L + j) * L, L)]
                            return jnp.where(_ji() == j, _splat(v, l), colv)

                        colv = lax.fori_loop(0, L, col_body, ninf)
                        hcnt = _fold_sum(
                            jnp.where(colv >= thr_vec, 1.0, 0.0)
                        )[0].astype(jnp.int32)

                        def emit1(c4, colv=colv):
                            cand[pl.ds(c4, L)] = _fold_max(colv)
                            return c4 + 1

                        def emitn(c4, colv=colv, hcnt=hcnt):
                            cand[pl.ds(c4, L)] = _sort16_desc(colv)
                            return c4 + hcnt

                        def emit(c4, hcnt=hcnt, emit1=emit1, emitn=emitn):
                            return lax.cond(hcnt == 1, emit1, emitn, c4)

                        c3 = lax.cond(hcnt > 0, emit, lambda c4: c4, c3)
                    return c3

                is_multi = _splat(cm2v, l)[0] >= thr_s
                return lax.cond(is_multi, multi, single, c2)

            def lane_iter(i, st):
                bits, c2 = st
                b = bits & (-bits)
                l = (lax.bitcast_convert_type(
                    b.astype(jnp.float32), jnp.int32) >> 23) - 127
                return (bits ^ b, lane_do(l, c2))

            return lax.fori_loop(0, nl, lane_iter, (bits0, c))[1]

        return lax.cond(bits0 != 0, hit_do, lambda cc: cc, c)

    cur = lax.fori_loop(0, NCM, cm_body, jnp.int32(0))

    cand[pl.ds(cur, L)] = ninf
    cand[pl.ds(cur + L, L)] = ninf

    nchunks = (cur + 2 * L - 1) // (2 * L)

    def p3_body(c, carry):
        r0, r1 = carry
        c0 = cand[pl.ds(c * 2 * L, L)]
        c1 = cand[pl.ds(c * 2 * L + L, L)]
        return _merge_chunk(r0, r1, c0, c1)

    return lax.fori_loop(0, nchunks, p3_body, (ninf, ninf))


def _sc_body(x_hbm, out_hbm, buf0, buf1, cm, cm2, cand, outb, sem0, sem1):
    wid = lax.axis_index("s") * NC + lax.axis_index("c")
    row0 = wid * RPW
    bufs = (buf0, buf1)
    sems = (sem0, sem1)

    pltpu.make_async_copy(x_hbm.at[row0], buf0, sem0).start()
    for r in range(RPW):
        buf, sem = bufs[r % 2], sems[r % 2]
        pltpu.make_async_copy(x_hbm.at[row0 + r], buf, sem).wait()
        if r + 1 < RPW:
            nbuf, nsem = bufs[(r + 1) % 2], sems[(r + 1) % 2]
            pltpu.make_async_copy(x_hbm.at[row0 + r + 1], nbuf, nsem).start()
        t0, t1 = _row_topk(buf, cm, cm2, cand)
        outb[r, pl.ds(0, L)] = t0
        outb[r, pl.ds(L, L)] = t1
    pltpu.sync_copy(outb, out_hbm.at[pl.ds(row0, RPW)])


def kernel(x):
    mesh = plsc.VectorSubcoreMesh(
        core_axis_name="c", subcore_axis_name="s", num_cores=NC,
        num_subcores=NS)
    run = pl.kernel(
        _sc_body,
        out_type=jax.ShapeDtypeStruct((ROWS, K), jnp.float32),
        mesh=mesh,
        scratch_types=[
            pltpu.VMEM((N,), jnp.float32),
            pltpu.VMEM((N,), jnp.float32),
            pltpu.VMEM((NCM * L,), jnp.float32),
            pltpu.VMEM((NCM * L,), jnp.float32),
            pltpu.VMEM((CAND,), jnp.float32),
            pltpu.VMEM((RPW, K), jnp.float32),
            pltpu.SemaphoreType.DMA,
            pltpu.SemaphoreType.DMA,
        ],
    )
    return run(x)
